# baseline (speedup 1.0000x reference)
import jax
import jax.numpy as jnp
from jax import lax
from jax.experimental import pallas as pl
from jax.experimental.pallas import tpu as pltpu
from jax.experimental.pallas import tpu_sc as plsc

N_NODES = 10000
N_EDGES = 320000
N_FEAT = 128
N_HID = 128
ALPHA_SLOPE = 0.05

NC = 2
NS = 16
L = 16

EPW1 = 20160
E_PAD = NS * EPW1
K = 80
CH = 1680
NCH1 = EPW1 // CH
NSC0 = 8
H_MAX = 8 * CH
NPAD = 10240
RPW = NPAD // NS


def _tc_prep_body(x_ref, wt_ref, ap_ref, ab_ref, z_ref, e_ref):
    zb = jnp.dot(x_ref[...], wt_ref[...], preferred_element_type=jnp.float32)
    z_ref[...] = zb
    e_ref[...] = jnp.dot(zb, ap_ref[...], preferred_element_type=jnp.float32) + ab_ref[0]


def _tc_add_body(a_ref, b_ref, o_ref):
    o_ref[...] = a_ref[...] + b_ref[...]


def _node2d(n16):
    return [lax.shift_right_logical(n16, 7), jnp.bitwise_and(n16, 127)]


def _sc_body(z_hbm, e1_hbm, e2_hbm, sd_hbm, alpha_hbm, parts_hbm,
             bufA, bufB, bufC, h_v, sd_v, idx_v, srow0, srow1, srow2,
             gsem0, gsem1, gsem2, ssem0, ssem1, ssem2, out_sh, hsum_sh):
    c = lax.axis_index("c")
    s = lax.axis_index("s")
    base1 = s * EPW1
    base2 = base1 + c * (NSC0 * CH)
    nsc = NSC0 - (NSC0 - (12 - NSC0)) * c

    zero16f = jnp.zeros((L,), jnp.float32)
    iota16 = lax.iota(jnp.int32, L)
    zi16 = jnp.zeros((L,), jnp.int32)

    bufs = (bufA, bufB, bufC)
    gsems = (gsem0, gsem1, gsem2)
    ssems = (ssem0, ssem1, ssem2)
    srows = (srow0, srow1, srow2)

    def zz(i, _):
        for j in range(N_HID // L):
            bufA[i, pl.ds(j * L, L)] = zero16f
            bufC[i, pl.ds(j * L, L)] = zero16f
        return 0
    lax.fori_loop(0, K, zz, 0)
    for kk in range(RPW // K):
        pltpu.async_copy(bufA, out_sh.at[pl.ds(s * RPW + kk * K, K)], gsem0)
    pltpu.sync_copy(bufC.at[pl.ds(s * 5, 5)], hsum_sh.at[pl.ds(s * 5, 5)])
    for kk in range(RPW // K):
        pltpu.make_async_copy(bufA, out_sh.at[pl.ds(s * RPW + kk * K, K)],
                              gsem0).wait()
    for j in range(K // L):
        idx_v[pl.ds(j * L, L)] = iota16 + j * L
    plsc.subcore_barrier()

    pltpu.sync_copy(e1_hbm, bufA)
    pltpu.sync_copy(e2_hbm, bufB)

    def p1c(cc, _):
        pltpu.sync_copy(sd_hbm.at[pl.ds((s * NCH1 + cc) * 2 * CH, 2 * CH)], sd_v)
        hoff = (cc - c * NSC0) * CH

        def p1(i, _):
            off = i * L
            s16 = sd_v[pl.ds(off, L)]
            d16 = sd_v[pl.ds(CH + off, L)]
            hv = (plsc.load_gather(bufA, _node2d(s16))
                  + plsc.load_gather(bufB, _node2d(d16)))
            hv = jnp.exp(jnp.maximum(hv, ALPHA_SLOPE * hv))

            @pl.when((cc >= c * NSC0) & (cc < c * NSC0 + nsc))
            def _():
                h_v[pl.ds(hoff + off, L)] = hv

            plsc.addupdate_scatter(bufC, _node2d(s16), hv)
            return 0
        lax.fori_loop(0, CH // L, p1, 0)
        return 0
    with jax.named_scope("p1_edges"):
        lax.fori_loop(0, NCH1, p1c, 0)

    with jax.named_scope("p1_combine"):
        pltpu.sync_copy(bufC, hsum_sh.at[idx_v], add=True)
        plsc.subcore_barrier()
        pltpu.sync_copy(hsum_sh, bufC)

    def pac(cc, _):
        pltpu.sync_copy(
            sd_hbm.at[pl.ds((s * NCH1 + c * NSC0 + cc) * 2 * CH, CH)],
            sd_v.at[pl.ds(0, CH)])

        def pa(i, _):
            off = cc * CH + i * L
            s16 = sd_v[pl.ds(i * L, L)]
            hs16 = plsc.load_gather(bufC, _node2d(s16))
            h_v[pl.ds(off, L)] = h_v[pl.ds(off, L)] / hs16
            return 0
        lax.fori_loop(0, CH // L, pa, 0)
        pltpu.sync_copy(h_v.at[pl.ds(cc * CH, CH)],
                        alpha_hbm.at[pl.ds(base2 + cc * CH, CH)])
        return 0
    with jax.named_scope("alpha_prepass"):
        lax.fori_loop(0, nsc, pac, 0)

    def _gd(t, off):
        return pltpu.make_async_copy(
            z_hbm.at[sd_v.at[pl.ds(CH + off, K)]], bufs[t], gsems[t])

    def _sd(t, buf):
        return pltpu.make_async_copy(buf, out_sh.at[srows[t]], ssems[t])

    def p2s(sc, _):
        pltpu.sync_copy(
            sd_hbm.at[pl.ds((s * NCH1 + c * NSC0 + sc) * 2 * CH, 2 * CH)], sd_v)

        for t in range(2):
            @pl.when(sc > 0)
            def _():
                _sd(t, bufs[t]).wait()
            _gd(t, t * K).start()

        def p2(m, _):
            for t in range(3):
                k = 3 * m + t
                buf, srow = bufs[t], srows[t]
                _gd(t, k * K).wait()
                for j in range(K // L):
                    srow[pl.ds(j * L, L)] = sd_v[pl.ds(k * K + j * L, L)]
                aoff = sc * CH + k * K

                def scale(e2, _):
                    e = 2 * e2
                    af0 = plsc.load_gather(h_v, [zi16 + (aoff + e)])
                    af1 = plsc.load_gather(h_v, [zi16 + (aoff + e + 1)])
                    for j in range(N_HID // L):
                        buf[e, pl.ds(j * L, L)] = buf[e, pl.ds(j * L, L)] * af0
                    for j in range(N_HID // L):
                        buf[e + 1, pl.ds(j * L, L)] = (
                            buf[e + 1, pl.ds(j * L, L)] * af1)
                    return 0
                lax.fori_loop(0, K // 2, scale, 0)
                _sd(t, buf).start(add=True)

                t2 = (t + 2) % 3
                kn = k + 2

                @pl.when(kn <= 20)
                def _():
                    @pl.when((sc + k) > 0)
                    def _():
                        _sd(t2, bufs[t2]).wait()
                    _gd(t2, kn * K).start()
            return 0
        lax.fori_loop(0, CH // K // 3, p2, 0)
        return 0
    with jax.named_scope("p2_rows"):
        lax.fori_loop(0, nsc, p2s, 0)

        for t in range(3):
            _sd(t, bufs[t]).wait()
    plsc.subcore_barrier()

    with jax.named_scope("copyout"):
        pltpu.sync_copy(out_sh.at[pl.ds(s * RPW, RPW)],
                        parts_hbm.at[c, pl.ds(s * RPW, RPW)])


@jax.jit
def _run(x, src, dst, Wt, Apad, ab2):
    z, e_pad = pl.pallas_call(
        _tc_prep_body,
        grid=(10,),
        in_specs=[
            pl.BlockSpec((1000, N_FEAT), lambda i: (i, 0)),
            pl.BlockSpec((N_FEAT, N_HID), lambda i: (0, 0)),
            pl.BlockSpec((N_HID, N_HID), lambda i: (0, 0)),
            pl.BlockSpec(memory_space=pltpu.SMEM),
        ],
        out_specs=[
            pl.BlockSpec((1000, N_HID), lambda i: (i, 0)),
            pl.BlockSpec((1000, N_HID), lambda i: (i, 0)),
        ],
        out_shape=[
            jax.ShapeDtypeStruct((N_NODES, N_HID), jnp.float32),
            jax.ShapeDtypeStruct((N_NODES, N_HID), jnp.float32),
        ],
    )(x, Wt, Apad, ab2)

    e1 = jnp.pad(e_pad[:, 0], (0, NPAD - N_NODES)).reshape(K, N_HID)
    e2 = jnp.pad(e_pad[:, 1], (0, NPAD - N_NODES)).reshape(K, N_HID)
    src_p = jnp.concatenate(
        [src, jnp.full((E_PAD - N_EDGES,), N_NODES, jnp.int32)])
    dst_p = jnp.concatenate(
        [dst, jnp.zeros((E_PAD - N_EDGES,), jnp.int32)])
    sd = jnp.stack([src_p.reshape(E_PAD // CH, CH),
                    dst_p.reshape(E_PAD // CH, CH)], axis=1).reshape(-1)

    sc = pl.kernel(
        _sc_body,
        out_type=(
            jax.ShapeDtypeStruct((E_PAD,), jnp.float32),
            jax.ShapeDtypeStruct((NC, NPAD, N_HID), jnp.float32),
        ),
        mesh=plsc.VectorSubcoreMesh(core_axis_name="c", subcore_axis_name="s"),
        compiler_params=pltpu.CompilerParams(
            needs_layout_passes=False, use_tc_tiling_on_sc=False),
        scratch_types=[
            pltpu.VMEM((K, N_HID), jnp.float32),
            pltpu.VMEM((K, N_HID), jnp.float32),
            pltpu.VMEM((K, N_HID), jnp.float32),
            pltpu.VMEM((H_MAX,), jnp.float32),
            pltpu.VMEM((2 * CH,), jnp.int32),
            pltpu.VMEM((K,), jnp.int32),
            pltpu.VMEM((K,), jnp.int32),
            pltpu.VMEM((K,), jnp.int32),
            pltpu.VMEM((K,), jnp.int32),
            pltpu.SemaphoreType.DMA,
            pltpu.SemaphoreType.DMA,
            pltpu.SemaphoreType.DMA,
            pltpu.SemaphoreType.DMA,
            pltpu.SemaphoreType.DMA,
            pltpu.SemaphoreType.DMA,
            pltpu.VMEM_SHARED((NPAD, N_HID), jnp.float32),
            pltpu.VMEM_SHARED((K, N_HID), jnp.float32),
        ],
    )
    alpha_p, parts = sc(z, e1, e2, sd)

    out = pl.pallas_call(
        _tc_add_body,
        grid=(10,),
        in_specs=[
            pl.BlockSpec((1000, N_HID), lambda i: (i, 0)),
            pl.BlockSpec((1000, N_HID), lambda i: (i, 0)),
        ],
        out_specs=pl.BlockSpec((1000, N_HID), lambda i: (i, 0)),
        out_shape=jax.ShapeDtypeStruct((N_NODES, N_HID), jnp.float32),
    )(parts[0, :N_NODES], parts[1, :N_NODES])
    return out, alpha_p[:N_EDGES]


def kernel(x, edge_index, W, a_w, a_b):
    ei = edge_index.astype(jnp.int32)
    src = ei[0]
    dst = ei[1]
    Wt = W.T
    a1 = a_w[0, :N_HID]
    a2 = a_w[0, N_HID:]
    Apad = jnp.zeros((N_HID, N_HID), jnp.float32).at[:, 0].set(a1).at[:, 1].set(a2)
    ab2 = (a_b * 0.5).astype(jnp.float32)
    return _run(x, src, dst, Wt, Apad, ab2)


# speedup vs baseline: 1.0080x; 1.0080x over previous
import jax
import jax.numpy as jnp
from jax import lax
from jax.experimental import pallas as pl
from jax.experimental.pallas import tpu as pltpu
from jax.experimental.pallas import tpu_sc as plsc

N_NODES = 10000
N_EDGES = 320000
N_FEAT = 128
N_HID = 128
ALPHA_SLOPE = 0.05

NC = 2
NS = 16
L = 16

EPW1 = 20160
E_PAD = NS * EPW1
K = 80
CH = 1680
NCH1 = EPW1 // CH
NSC0 = 8
H_MAX = 8 * CH
NPAD = 10240
RPW = NPAD // NS


def _tc_prep_body(x_ref, wt_ref, ap_ref, ab_ref, z_ref, e_ref):
    zb = jnp.dot(x_ref[...], wt_ref[...], preferred_element_type=jnp.float32)
    z_ref[...] = zb
    e_ref[...] = jnp.dot(zb, ap_ref[...], preferred_element_type=jnp.float32) + ab_ref[0]


def _tc_add_body(a_ref, b_ref, o_ref):
    o_ref[...] = a_ref[...] + b_ref[...]


def _node2d(n16):
    return [lax.shift_right_logical(n16, 7), jnp.bitwise_and(n16, 127)]


def _sc_body(z_hbm, e1_hbm, e2_hbm, sd_hbm, alpha_hbm, parts_hbm,
             bufA, bufB, bufC, h_v, sd_v, idx_v, srow0, srow1, srow2,
             gsem0, gsem1, gsem2, ssem0, ssem1, ssem2, out_sh, hsum_sh):
    c = lax.axis_index("c")
    s = lax.axis_index("s")
    base1 = s * EPW1
    base2 = base1 + c * (NSC0 * CH)
    nsc = NSC0 - (NSC0 - (12 - NSC0)) * c

    zero16f = jnp.zeros((L,), jnp.float32)
    iota16 = lax.iota(jnp.int32, L)
    zi16 = jnp.zeros((L,), jnp.int32)

    bufs = (bufA, bufB, bufC)
    gsems = (gsem0, gsem1, gsem2)
    ssems = (ssem0, ssem1, ssem2)
    srows = (srow0, srow1, srow2)

    def zz(i, _):
        for j in range(N_HID // L):
            bufA[i, pl.ds(j * L, L)] = zero16f
            bufC[i, pl.ds(j * L, L)] = zero16f
        return 0
    lax.fori_loop(0, K, zz, 0)
    for kk in range(RPW // K):
        pltpu.async_copy(bufA, out_sh.at[pl.ds(s * RPW + kk * K, K)], gsem0)
    pltpu.sync_copy(bufC.at[pl.ds(s * 5, 5)], hsum_sh.at[pl.ds(s * 5, 5)])
    for kk in range(RPW // K):
        pltpu.make_async_copy(bufA, out_sh.at[pl.ds(s * RPW + kk * K, K)],
                              gsem0).wait()
    for j in range(K // L):
        idx_v[pl.ds(j * L, L)] = iota16 + j * L
    plsc.subcore_barrier()

    pltpu.sync_copy(e1_hbm, bufA)
    pltpu.sync_copy(e2_hbm, bufB)

    def p1c(cc, _):
        pltpu.sync_copy(sd_hbm.at[pl.ds((s * NCH1 + cc) * 2 * CH, 2 * CH)], sd_v)
        hoff = (cc - c * NSC0) * CH

        def p1(i, _):
            off = i * L
            s16 = sd_v[pl.ds(off, L)]
            d16 = sd_v[pl.ds(CH + off, L)]
            hv = (plsc.load_gather(bufA, _node2d(s16))
                  + plsc.load_gather(bufB, _node2d(d16)))
            hv = jnp.exp(jnp.maximum(hv, ALPHA_SLOPE * hv))

            @pl.when((cc >= c * NSC0) & (cc < c * NSC0 + nsc))
            def _():
                h_v[pl.ds(hoff + off, L)] = hv

            plsc.addupdate_scatter(bufC, _node2d(s16), hv)
            return 0
        lax.fori_loop(0, CH // L, p1, 0)
        return 0
    with jax.named_scope("p1_edges"):
        lax.fori_loop(0, NCH1, p1c, 0)

    with jax.named_scope("p1_combine"):
        pltpu.sync_copy(bufC, hsum_sh.at[idx_v], add=True)
        plsc.subcore_barrier()
        pltpu.sync_copy(hsum_sh, bufC)

    def pac(cc, _):
        pltpu.sync_copy(
            sd_hbm.at[pl.ds((s * NCH1 + c * NSC0 + cc) * 2 * CH, CH)],
            sd_v.at[pl.ds(0, CH)])

        def pa(i, _):
            off = cc * CH + i * L
            s16 = sd_v[pl.ds(i * L, L)]
            hs16 = plsc.load_gather(bufC, _node2d(s16))
            h_v[pl.ds(off, L)] = h_v[pl.ds(off, L)] / hs16
            return 0
        lax.fori_loop(0, CH // L, pa, 0)
        pltpu.sync_copy(h_v.at[pl.ds(cc * CH, CH)],
                        alpha_hbm.at[pl.ds(base2 + cc * CH, CH)])
        return 0
    with jax.named_scope("alpha_prepass"):
        lax.fori_loop(0, nsc, pac, 0)

    def _gd(t, off):
        return pltpu.make_async_copy(
            z_hbm.at[sd_v.at[pl.ds(CH + off, K)]], bufs[t], gsems[t])

    def _sd(t, buf):
        return pltpu.make_async_copy(buf, out_sh.at[srows[t]], ssems[t])

    def p2s(sc, _):
        pltpu.sync_copy(
            sd_hbm.at[pl.ds((s * NCH1 + c * NSC0 + sc) * 2 * CH, 2 * CH)], sd_v)

        for t in range(2):
            @pl.when(sc > 0)
            def _():
                _sd(t, bufs[t]).wait()
            _gd(t, t * K).start()

        def p2(m, _):
            for t in range(3):
                k = 3 * m + t
                buf, srow = bufs[t], srows[t]
                _gd(t, k * K).wait()
                for j in range(K // L):
                    srow[pl.ds(j * L, L)] = sd_v[pl.ds(k * K + j * L, L)]
                aoff = sc * CH + k * K

                def scale(e4, _):
                    e = 4 * e4
                    afs = [plsc.load_gather(h_v, [zi16 + (aoff + e + d)])
                           for d in range(4)]
                    for d in range(4):
                        for j in range(N_HID // L):
                            buf[e + d, pl.ds(j * L, L)] = (
                                buf[e + d, pl.ds(j * L, L)] * afs[d])
                    return 0
                lax.fori_loop(0, K // 4, scale, 0)
                _sd(t, buf).start(add=True)

                t2 = (t + 2) % 3
                kn = k + 2

                @pl.when(kn <= 20)
                def _():
                    @pl.when((sc + k) > 0)
                    def _():
                        _sd(t2, bufs[t2]).wait()
                    _gd(t2, kn * K).start()
            return 0
        lax.fori_loop(0, CH // K // 3, p2, 0)
        return 0
    with jax.named_scope("p2_rows"):
        lax.fori_loop(0, nsc, p2s, 0)

        for t in range(3):
            _sd(t, bufs[t]).wait()
    plsc.subcore_barrier()

    with jax.named_scope("copyout"):
        pltpu.sync_copy(out_sh.at[pl.ds(s * RPW, RPW)],
                        parts_hbm.at[c, pl.ds(s * RPW, RPW)])


@jax.jit
def _run(x, src, dst, Wt, Apad, ab2):
    z, e_pad = pl.pallas_call(
        _tc_prep_body,
        grid=(10,),
        in_specs=[
            pl.BlockSpec((1000, N_FEAT), lambda i: (i, 0)),
            pl.BlockSpec((N_FEAT, N_HID), lambda i: (0, 0)),
            pl.BlockSpec((N_HID, N_HID), lambda i: (0, 0)),
            pl.BlockSpec(memory_space=pltpu.SMEM),
        ],
        out_specs=[
            pl.BlockSpec((1000, N_HID), lambda i: (i, 0)),
            pl.BlockSpec((1000, N_HID), lambda i: (i, 0)),
        ],
        out_shape=[
            jax.ShapeDtypeStruct((N_NODES, N_HID), jnp.float32),
            jax.ShapeDtypeStruct((N_NODES, N_HID), jnp.float32),
        ],
    )(x, Wt, Apad, ab2)

    e1 = jnp.pad(e_pad[:, 0], (0, NPAD - N_NODES)).reshape(K, N_HID)
    e2 = jnp.pad(e_pad[:, 1], (0, NPAD - N_NODES)).reshape(K, N_HID)
    src_p = jnp.concatenate(
        [src, jnp.full((E_PAD - N_EDGES,), N_NODES, jnp.int32)])
    dst_p = jnp.concatenate(
        [dst, jnp.zeros((E_PAD - N_EDGES,), jnp.int32)])
    sd = jnp.stack([src_p.reshape(E_PAD // CH, CH),
                    dst_p.reshape(E_PAD // CH, CH)], axis=1).reshape(-1)

    sc = pl.kernel(
        _sc_body,
        out_type=(
            jax.ShapeDtypeStruct((E_PAD,), jnp.float32),
            jax.ShapeDtypeStruct((NC, NPAD, N_HID), jnp.float32),
        ),
        mesh=plsc.VectorSubcoreMesh(core_axis_name="c", subcore_axis_name="s"),
        compiler_params=pltpu.CompilerParams(
            needs_layout_passes=False, use_tc_tiling_on_sc=False),
        scratch_types=[
            pltpu.VMEM((K, N_HID), jnp.float32),
            pltpu.VMEM((K, N_HID), jnp.float32),
            pltpu.VMEM((K, N_HID), jnp.float32),
            pltpu.VMEM((H_MAX,), jnp.float32),
            pltpu.VMEM((2 * CH,), jnp.int32),
            pltpu.VMEM((K,), jnp.int32),
            pltpu.VMEM((K,), jnp.int32),
            pltpu.VMEM((K,), jnp.int32),
            pltpu.VMEM((K,), jnp.int32),
            pltpu.SemaphoreType.DMA,
            pltpu.SemaphoreType.DMA,
            pltpu.SemaphoreType.DMA,
            pltpu.SemaphoreType.DMA,
            pltpu.SemaphoreType.DMA,
            pltpu.SemaphoreType.DMA,
            pltpu.VMEM_SHARED((NPAD, N_HID), jnp.float32),
            pltpu.VMEM_SHARED((K, N_HID), jnp.float32),
        ],
    )
    alpha_p, parts = sc(z, e1, e2, sd)

    out = pl.pallas_call(
        _tc_add_body,
        grid=(10,),
        in_specs=[
            pl.BlockSpec((1000, N_HID), lambda i: (i, 0)),
            pl.BlockSpec((1000, N_HID), lambda i: (i, 0)),
        ],
        out_specs=pl.BlockSpec((1000, N_HID), lambda i: (i, 0)),
        out_shape=jax.ShapeDtypeStruct((N_NODES, N_HID), jnp.float32),
    )(parts[0, :N_NODES], parts[1, :N_NODES])
    return out, alpha_p[:N_EDGES]


def kernel(x, edge_index, W, a_w, a_b):
    ei = edge_index.astype(jnp.int32)
    src = ei[0]
    dst = ei[1]
    Wt = W.T
    a1 = a_w[0, :N_HID]
    a2 = a_w[0, N_HID:]
    Apad = jnp.zeros((N_HID, N_HID), jnp.float32).at[:, 0].set(a1).at[:, 1].set(a2)
    ab2 = (a_b * 0.5).astype(jnp.float32)
    return _run(x, src, dst, Wt, Apad, ab2)


# speedup vs baseline: 1.0423x; 1.0340x over previous
import jax
import jax.numpy as jnp
from jax import lax
from jax.experimental import pallas as pl
from jax.experimental.pallas import tpu as pltpu
from jax.experimental.pallas import tpu_sc as plsc

N_NODES = 10000
N_EDGES = 320000
N_FEAT = 128
N_HID = 128
ALPHA_SLOPE = 0.05

NC = 2
NS = 16
L = 16

EPW1 = 20160
E_PAD = NS * EPW1
K = 80
CH = 3360
NCH1 = EPW1 // CH
NSC0 = 4
NSC1 = 6 - NSC0
NPAD = 10240
RPW = NPAD // NS


def _tc_prep_body(x_ref, wt_ref, ap_ref, ab_ref, z_ref, e_ref):
    zb = jnp.dot(x_ref[...], wt_ref[...], preferred_element_type=jnp.float32)
    z_ref[...] = zb
    e_ref[...] = jnp.dot(zb, ap_ref[...], preferred_element_type=jnp.float32) + ab_ref[0]


def _tc_add_body(a_ref, b_ref, o_ref):
    o_ref[...] = a_ref[...] + b_ref[...]


def _node2d(n16):
    return [lax.shift_right_logical(n16, 7), jnp.bitwise_and(n16, 127)]


def _sc_body(z_hbm, e1_hbm, e2_hbm, sd_hbm, alpha_hbm, parts_hbm,
             bufA, bufB, bufC, h_v, sd_v, idx_v, srow0, srow1, srow2,
             gsem0, gsem1, gsem2, ssem0, ssem1, ssem2, out_sh, hsum_sh):
    c = lax.axis_index("c")
    s = lax.axis_index("s")
    base1 = s * EPW1
    base2 = base1 + c * (NSC0 * CH)
    nsc = NSC0 - (NSC0 - NSC1) * c

    zero16f = jnp.zeros((L,), jnp.float32)
    iota16 = lax.iota(jnp.int32, L)
    zi16 = jnp.zeros((L,), jnp.int32)

    bufs = (bufA, bufB, bufC)
    gsems = (gsem0, gsem1, gsem2)
    ssems = (ssem0, ssem1, ssem2)
    srows = (srow0, srow1, srow2)

    def zz(i, _):
        for j in range(N_HID // L):
            bufA[i, pl.ds(j * L, L)] = zero16f
            bufC[i, pl.ds(j * L, L)] = zero16f
        return 0
    lax.fori_loop(0, K, zz, 0)
    for kk in range(RPW // K):
        pltpu.async_copy(bufA, out_sh.at[pl.ds(s * RPW + kk * K, K)], gsem0)
    pltpu.sync_copy(bufC.at[pl.ds(s * 5, 5)], hsum_sh.at[pl.ds(s * 5, 5)])
    for kk in range(RPW // K):
        pltpu.make_async_copy(bufA, out_sh.at[pl.ds(s * RPW + kk * K, K)],
                              gsem0).wait()
    for j in range(K // L):
        idx_v[pl.ds(j * L, L)] = iota16 + j * L
    plsc.subcore_barrier()

    pltpu.sync_copy(e1_hbm, bufA)
    pltpu.sync_copy(e2_hbm, bufB)

    def p1c(cc, _):
        pltpu.sync_copy(sd_hbm.at[pl.ds((s * NCH1 + cc) * 2 * CH, 2 * CH)], sd_v)
        def p1(i, _):
            off = i * L
            s16 = sd_v[pl.ds(off, L)]
            d16 = sd_v[pl.ds(CH + off, L)]
            hv = (plsc.load_gather(bufA, _node2d(s16))
                  + plsc.load_gather(bufB, _node2d(d16)))
            hv = jnp.exp(jnp.maximum(hv, ALPHA_SLOPE * hv))
            h_v[pl.ds(off, L)] = hv
            plsc.addupdate_scatter(bufC, _node2d(s16), hv)
            return 0
        lax.fori_loop(0, CH // L, p1, 0)

        @pl.when((cc >= c * NSC0) & (cc < c * NSC0 + nsc))
        def _():
            pltpu.sync_copy(h_v, alpha_hbm.at[pl.ds(base1 + cc * CH, CH)])
        return 0
    with jax.named_scope("p1_edges"):
        lax.fori_loop(0, NCH1, p1c, 0)

    with jax.named_scope("p1_combine"):
        pltpu.sync_copy(bufC, hsum_sh.at[idx_v], add=True)
        plsc.subcore_barrier()
        pltpu.sync_copy(hsum_sh, bufC)

    def pac(cc, _):
        pltpu.sync_copy(
            sd_hbm.at[pl.ds((s * NCH1 + c * NSC0 + cc) * 2 * CH, CH)],
            sd_v.at[pl.ds(0, CH)])
        pltpu.sync_copy(alpha_hbm.at[pl.ds(base2 + cc * CH, CH)], h_v)

        def pa(i, _):
            off = i * L
            s16 = sd_v[pl.ds(off, L)]
            hs16 = plsc.load_gather(bufC, _node2d(s16))
            h_v[pl.ds(off, L)] = h_v[pl.ds(off, L)] / hs16
            return 0
        lax.fori_loop(0, CH // L, pa, 0)
        pltpu.sync_copy(h_v, alpha_hbm.at[pl.ds(base2 + cc * CH, CH)])
        return 0
    with jax.named_scope("alpha_prepass"):
        lax.fori_loop(0, nsc, pac, 0)

    def _gd(t, off):
        return pltpu.make_async_copy(
            z_hbm.at[sd_v.at[pl.ds(CH + off, K)]], bufs[t], gsems[t])

    def _sd(t, buf):
        return pltpu.make_async_copy(buf, out_sh.at[srows[t]], ssems[t])

    def p2s(sc, _):
        pltpu.sync_copy(
            sd_hbm.at[pl.ds((s * NCH1 + c * NSC0 + sc) * 2 * CH, 2 * CH)], sd_v)
        pltpu.sync_copy(alpha_hbm.at[pl.ds(base2 + sc * CH, CH)], h_v)

        for t in range(2):
            @pl.when(sc > 0)
            def _():
                _sd(t, bufs[t]).wait()
            _gd(t, t * K).start()

        def p2(m, _):
            for t in range(3):
                k = 3 * m + t
                buf, srow = bufs[t], srows[t]
                _gd(t, k * K).wait()
                for j in range(K // L):
                    srow[pl.ds(j * L, L)] = sd_v[pl.ds(k * K + j * L, L)]
                aoff = k * K

                def scale(e4, _):
                    e = 4 * e4
                    afs = [plsc.load_gather(h_v, [zi16 + (aoff + e + d)])
                           for d in range(4)]
                    for d in range(4):
                        for j in range(N_HID // L):
                            buf[e + d, pl.ds(j * L, L)] = (
                                buf[e + d, pl.ds(j * L, L)] * afs[d])
                    return 0
                lax.fori_loop(0, K // 4, scale, 0)
                _sd(t, buf).start(add=True)

                t2 = (t + 2) % 3
                kn = k + 2

                @pl.when(kn <= CH // K - 1)
                def _():
                    @pl.when((sc + k) > 0)
                    def _():
                        _sd(t2, bufs[t2]).wait()
                    _gd(t2, kn * K).start()
            return 0
        lax.fori_loop(0, CH // K // 3, p2, 0)
        return 0
    with jax.named_scope("p2_rows"):
        lax.fori_loop(0, nsc, p2s, 0)

        for t in range(3):
            _sd(t, bufs[t]).wait()
    plsc.subcore_barrier()

    with jax.named_scope("copyout"):
        pltpu.sync_copy(out_sh.at[pl.ds(s * RPW, RPW)],
                        parts_hbm.at[c, pl.ds(s * RPW, RPW)])


@jax.jit
def _run(x, src, dst, Wt, Apad, ab2):
    z, e_pad = pl.pallas_call(
        _tc_prep_body,
        grid=(10,),
        in_specs=[
            pl.BlockSpec((1000, N_FEAT), lambda i: (i, 0)),
            pl.BlockSpec((N_FEAT, N_HID), lambda i: (0, 0)),
            pl.BlockSpec((N_HID, N_HID), lambda i: (0, 0)),
            pl.BlockSpec(memory_space=pltpu.SMEM),
        ],
        out_specs=[
            pl.BlockSpec((1000, N_HID), lambda i: (i, 0)),
            pl.BlockSpec((1000, N_HID), lambda i: (i, 0)),
        ],
        out_shape=[
            jax.ShapeDtypeStruct((N_NODES, N_HID), jnp.float32),
            jax.ShapeDtypeStruct((N_NODES, N_HID), jnp.float32),
        ],
    )(x, Wt, Apad, ab2)

    e1 = jnp.pad(e_pad[:, 0], (0, NPAD - N_NODES)).reshape(K, N_HID)
    e2 = jnp.pad(e_pad[:, 1], (0, NPAD - N_NODES)).reshape(K, N_HID)
    src_p = jnp.concatenate(
        [src, jnp.full((E_PAD - N_EDGES,), N_NODES, jnp.int32)])
    dst_p = jnp.concatenate(
        [dst, jnp.zeros((E_PAD - N_EDGES,), jnp.int32)])
    sd = jnp.stack([src_p.reshape(E_PAD // CH, CH),
                    dst_p.reshape(E_PAD // CH, CH)], axis=1).reshape(-1)

    sc = pl.kernel(
        _sc_body,
        out_type=(
            jax.ShapeDtypeStruct((E_PAD,), jnp.float32),
            jax.ShapeDtypeStruct((NC, NPAD, N_HID), jnp.float32),
        ),
        mesh=plsc.VectorSubcoreMesh(core_axis_name="c", subcore_axis_name="s"),
        compiler_params=pltpu.CompilerParams(
            needs_layout_passes=False, use_tc_tiling_on_sc=False),
        scratch_types=[
            pltpu.VMEM((K, N_HID), jnp.float32),
            pltpu.VMEM((K, N_HID), jnp.float32),
            pltpu.VMEM((K, N_HID), jnp.float32),
            pltpu.VMEM((CH,), jnp.float32),
            pltpu.VMEM((2 * CH,), jnp.int32),
            pltpu.VMEM((K,), jnp.int32),
            pltpu.VMEM((K,), jnp.int32),
            pltpu.VMEM((K,), jnp.int32),
            pltpu.VMEM((K,), jnp.int32),
            pltpu.SemaphoreType.DMA,
            pltpu.SemaphoreType.DMA,
            pltpu.SemaphoreType.DMA,
            pltpu.SemaphoreType.DMA,
            pltpu.SemaphoreType.DMA,
            pltpu.SemaphoreType.DMA,
            pltpu.VMEM_SHARED((NPAD, N_HID), jnp.float32),
            pltpu.VMEM_SHARED((K, N_HID), jnp.float32),
        ],
    )
    alpha_p, parts = sc(z, e1, e2, sd)

    out = pl.pallas_call(
        _tc_add_body,
        grid=(10,),
        in_specs=[
            pl.BlockSpec((1000, N_HID), lambda i: (i, 0)),
            pl.BlockSpec((1000, N_HID), lambda i: (i, 0)),
        ],
        out_specs=pl.BlockSpec((1000, N_HID), lambda i: (i, 0)),
        out_shape=jax.ShapeDtypeStruct((N_NODES, N_HID), jnp.float32),
    )(parts[0, :N_NODES], parts[1, :N_NODES])
    return out, alpha_p[:N_EDGES]


def kernel(x, edge_index, W, a_w, a_b):
    ei = edge_index.astype(jnp.int32)
    src = ei[0]
    dst = ei[1]
    Wt = W.T
    a1 = a_w[0, :N_HID]
    a2 = a_w[0, N_HID:]
    Apad = jnp.zeros((N_HID, N_HID), jnp.float32).at[:, 0].set(a1).at[:, 1].set(a2)
    ab2 = (a_b * 0.5).astype(jnp.float32)
    return _run(x, src, dst, Wt, Apad, ab2)
